# Initial kernel scaffold; baseline (speedup 1.0000x reference)
import jax
import jax.numpy as jnp
from jax.experimental import pallas as pl


def kernel(x, W1, g1, b1, W2, g2, b2, W3, g3, b3, W4, g4, b4, W5, g5, b5, W6, bias6, g6, b6, W7, bias7, g7, b7, W8, bias8):
    raise NotImplementedError("write your pallas kernel here")


# speedup vs baseline: 2.4204x; 2.4204x over previous
import functools

import jax
import jax.numpy as jnp
from jax import lax
from jax.experimental import pallas as pl
from jax.experimental.pallas import tpu as pltpu
from jax.experimental.pallas import tpu_sc as plsc

B = 16
N = 1024
C = 64
K = 20
EPS = 1e-5
NEG = -3e38
R = 256
NR = N // R
HALF = N // 2


def _topk_body(d_ref, idx_ref):
    d = d_ref[0]
    iota = lax.broadcasted_iota(jnp.int32, (R, N), 1)
    for t in range(K):
        rowmax = jnp.max(d, axis=1, keepdims=True)
        cand = jnp.where(d == rowmax, iota, jnp.int32(2**30))
        amin = jnp.min(cand, axis=1, keepdims=True)
        idx_ref[0, :, pl.ds(t, 1)] = amin
        d = jnp.where(cand == amin, NEG, d)


def _topk_from_dist(dist):
    return pl.pallas_call(
        _topk_body,
        grid=(B, NR),
        in_specs=[pl.BlockSpec((1, R, N), lambda b, r: (b, r, 0))],
        out_specs=pl.BlockSpec((1, R, K), lambda b, r: (b, r, 0)),
        out_shape=jax.ShapeDtypeStruct((B, N, K), jnp.int32),
    )(dist)


def _knn_idx(h):
    inner = -2.0 * jnp.einsum('bcn,bcm->bnm', h, h)
    xx = jnp.sum(h * h, axis=1)
    dist = -xx[:, :, None] - inner - xx[:, None, :]
    return _topk_from_dist(dist)


def _conv_body(h_ref, w_ref, y_ref):
    y_ref[0] = jnp.dot(w_ref[...], h_ref[0], preferred_element_type=jnp.float32)


def _conv(h, W):
    return pl.pallas_call(
        _conv_body,
        grid=(B,),
        in_specs=[
            pl.BlockSpec((1, C, N), lambda b: (b, 0, 0)),
            pl.BlockSpec((C, C), lambda b: (0, 0)),
        ],
        out_specs=pl.BlockSpec((1, C, N), lambda b: (b, 0, 0)),
        out_shape=jax.ShapeDtypeStruct((B, C, N), jnp.float32),
    )(h, W)


@functools.partial(
    pl.kernel,
    mesh=plsc.VectorSubcoreMesh(core_axis_name="c", subcore_axis_name="s"),
    compiler_params=pltpu.CompilerParams(needs_layout_passes=False),
    out_type=(
        jax.ShapeDtypeStruct((B, 2, C * HALF), jnp.float32),
        jax.ShapeDtypeStruct((32, 2, C * 16), jnp.float32),
    ),
    scratch_types=[
        pltpu.VMEM((C * N,), jnp.float32),
        pltpu.VMEM((K * HALF,), jnp.int32),
        pltpu.VMEM((C * HALF,), jnp.float32),
        pltpu.VMEM((C * 16,), jnp.float32),
        pltpu.VMEM((C * 16,), jnp.float32),
    ],
)
def _sc_gathermax(y_hbm, idx_hbm, m_hbm, sp_hbm, yv, idxv, mv, s1v, s2v):
    b = lax.axis_index("s")
    half = lax.axis_index("c")
    wid = b * 2 + half
    pltpu.sync_copy(y_hbm.at[b], yv)
    pltpu.sync_copy(idx_hbm.at[b, half], idxv)

    def z_body(c, carry):
        off = pl.multiple_of(c * 16, 16)
        s1v[pl.ds(off, 16)] = jnp.zeros((16,), jnp.float32)
        s2v[pl.ds(off, 16)] = jnp.zeros((16,), jnp.float32)
        return carry

    lax.fori_loop(0, C, z_body, 0)

    def n0_body(n0, carry):
        nb = pl.multiple_of(n0 * 16, 16)
        ridx = [idxv[pl.ds(k * HALF + nb, 16)] for k in range(K)]

        def c_body(c, inner):
            cbase = jnp.full((16,), c * N, jnp.int32)
            acc = jnp.full((16,), NEG, jnp.float32)
            s1 = jnp.zeros((16,), jnp.float32)
            s2 = jnp.zeros((16,), jnp.float32)
            for k in range(K):
                v = plsc.load_gather(yv, [cbase + ridx[k]])
                acc = jnp.maximum(acc, v)
                s1 = s1 + v
                s2 = s2 + v * v
            mv[pl.ds(pl.multiple_of(c * HALF, 16) + nb, 16)] = acc
            soff = pl.multiple_of(c * 16, 16)
            s1v[pl.ds(soff, 16)] = s1v[pl.ds(soff, 16)] + s1
            s2v[pl.ds(soff, 16)] = s2v[pl.ds(soff, 16)] + s2
            return inner

        return lax.fori_loop(0, C, c_body, carry)

    lax.fori_loop(0, HALF // 16, n0_body, 0)
    pltpu.sync_copy(mv, m_hbm.at[b, half])
    pltpu.sync_copy(s1v, sp_hbm.at[wid, 0])
    pltpu.sync_copy(s2v, sp_hbm.at[wid, 1])


NTOT = float(B * N * K)


def _norm_body(m_ref, s1_ref, s2_ref, g_ref, b_ref, o_ref):
    s1 = jnp.sum(s1_ref[...], axis=1, keepdims=True)
    s2 = jnp.sum(s2_ref[...], axis=1, keepdims=True)
    mean = s1 / NTOT
    var = s2 / NTOT - mean * mean
    sc = g_ref[...] / jnp.sqrt(var + EPS)
    sh = b_ref[...] - mean * sc
    o_ref[0] = jnp.maximum(m_ref[0] * sc + sh, 0.0)


def _norm(m, s1p, s2p, g, b):
    return pl.pallas_call(
        _norm_body,
        grid=(B,),
        in_specs=[
            pl.BlockSpec((1, C, N), lambda i: (i, 0, 0)),
            pl.BlockSpec((C, 512), lambda i: (0, 0)),
            pl.BlockSpec((C, 512), lambda i: (0, 0)),
            pl.BlockSpec((C, 1), lambda i: (0, 0)),
            pl.BlockSpec((C, 1), lambda i: (0, 0)),
        ],
        out_specs=pl.BlockSpec((1, C, N), lambda i: (i, 0, 0)),
        out_shape=jax.ShapeDtypeStruct((B, C, N), jnp.float32),
    )(m, s1p, s2p, g, b)


def _layer5_body(h_ref, w_ref, cm_ref, s1_ref, s2_ref):
    z = jnp.dot(w_ref[...], h_ref[0], preferred_element_type=jnp.float32)
    cm_ref[0] = jnp.max(z, axis=1, keepdims=True)
    s1_ref[0] = jnp.sum(z, axis=1, keepdims=True)
    s2_ref[0] = jnp.sum(z * z, axis=1, keepdims=True)


def _layer5(h, W5):
    shp = jax.ShapeDtypeStruct((B, 1024, 1), jnp.float32)
    return pl.pallas_call(
        _layer5_body,
        grid=(B,),
        in_specs=[
            pl.BlockSpec((1, C, N), lambda b: (b, 0, 0)),
            pl.BlockSpec((1024, C), lambda b: (0, 0)),
        ],
        out_specs=[pl.BlockSpec((1, 1024, 1), lambda b: (b, 0, 0))] * 3,
        out_shape=[shp, shp, shp],
    )(h, W5)


def _head_body(cm_ref, s1_ref, s2_ref, g5_ref, b5_ref, w6_ref, g6_ref, b6_ref,
               w7_ref, g7_ref, b7_ref, w8_ref, bias8_ref, o_ref):
    n5 = float(B * N)
    s1 = jnp.sum(s1_ref[...], axis=0, keepdims=True)
    s2 = jnp.sum(s2_ref[...], axis=0, keepdims=True)
    mean = s1 / n5
    var = s2 / n5 - mean * mean
    sc5 = g5_ref[...] / jnp.sqrt(var + EPS)
    sh5 = b5_ref[...] - mean * sc5
    h5 = jnp.maximum(cm_ref[...] * sc5 + sh5, 0.0)

    a = jnp.dot(h5, w6_ref[...], preferred_element_type=jnp.float32)
    m6 = jnp.mean(a, axis=0, keepdims=True)
    v6 = jnp.mean((a - m6) * (a - m6), axis=0, keepdims=True)
    h6 = jnp.maximum((a - m6) / jnp.sqrt(v6 + EPS) * g6_ref[...] + b6_ref[...], 0.0)

    a = jnp.dot(h6, w7_ref[...], preferred_element_type=jnp.float32)
    m7 = jnp.mean(a, axis=0, keepdims=True)
    v7 = jnp.mean((a - m7) * (a - m7), axis=0, keepdims=True)
    h7 = jnp.maximum((a - m7) / jnp.sqrt(v7 + EPS) * g7_ref[...] + b7_ref[...], 0.0)

    o_ref[...] = jnp.maximum(
        jnp.dot(h7, w8_ref[...], preferred_element_type=jnp.float32)
        + bias8_ref[...], 0.0)


def _head(cm, s1, s2, g5, b5, W6T, g6, b6, W7T, g7, b7, W8T, bias8):
    return pl.pallas_call(
        _head_body,
        out_shape=jax.ShapeDtypeStruct((B, 40), jnp.float32),
    )(cm, s1, s2, g5, b5, W6T, g6, b6, W7T, g7, b7, W8T, bias8)


def _edge_layer_exact(h, W, g, b):
    idx = _knn_idx(h)
    x_t = jnp.transpose(h, (0, 2, 1))
    feat = jax.vmap(lambda xt, id_: xt[id_])(x_t, idx)
    feat = jnp.transpose(feat, (0, 3, 2, 1))
    z = jnp.einsum('oc,bckn->bokn', W, feat)
    mean = jnp.mean(z, axis=(0, 2, 3), keepdims=True)
    var = jnp.var(z, axis=(0, 2, 3), keepdims=True)
    zn = (z - mean) / jnp.sqrt(var + EPS) * g[None, :, None, None] + b[None, :, None, None]
    return jnp.max(jax.nn.relu(zn), axis=2)


def _edge_layer(h, W, g, b):
    idx = _knn_idx(h)
    idx4 = jnp.transpose(idx.reshape(B, 2, HALF, K), (0, 1, 3, 2)).reshape(B, 2, K * HALF)
    y = _conv(h, W)
    m4, sp = _sc_gathermax(y.reshape(B, C * N), idx4)
    m = jnp.transpose(m4.reshape(B, 2, C, HALF), (0, 2, 1, 3)).reshape(B, C, N)
    s1p = jnp.transpose(sp[:, 0].reshape(32, C, 16), (1, 0, 2)).reshape(C, 512)
    s2p = jnp.transpose(sp[:, 1].reshape(32, C, 16), (1, 0, 2)).reshape(C, 512)
    return _norm(m, s1p, s2p, g.reshape(C, 1), b.reshape(C, 1))


def kernel(x, W1, g1, b1, W2, g2, b2, W3, g3, b3, W4, g4, b4, W5, g5, b5,
           W6, bias6, g6, b6, W7, bias7, g7, b7, W8, bias8):
    h = _edge_layer_exact(x, W1, g1, b1)
    h = _edge_layer_exact(h, W2, g2, b2)
    h = _edge_layer_exact(h, W3, g3, b3)
    h = _edge_layer(h, W4, g4, b4)
    cm, s1, s2 = _layer5(h, W5)
    out = _head(
        cm.reshape(B, 1024), s1.reshape(B, 1024), s2.reshape(B, 1024),
        g5.reshape(1, 1024), b5.reshape(1, 1024),
        W6.T, g6.reshape(1, 512), (b6 + 0.0 * bias6).reshape(1, 512),
        W7.T, g7.reshape(1, 128), (b7 + 0.0 * bias7).reshape(1, 128),
        W8.T, bias8.reshape(1, 40))
    return out


# speedup vs baseline: 2.4696x; 1.0204x over previous
import functools

import jax
import jax.numpy as jnp
from jax import lax
from jax.experimental import pallas as pl
from jax.experimental.pallas import tpu as pltpu
from jax.experimental.pallas import tpu_sc as plsc

B = 16
N = 1024
C = 64
K = 20
EPS = 1e-5
NEG = -3e38
R = 512
NR = N // R
HALF = N // 2


def _topk_body(d_ref, idx_ref):
    d = d_ref[0]
    iota = lax.broadcasted_iota(jnp.int32, (R, N), 1)
    for t in range(K):
        rowmax = jnp.max(d, axis=1, keepdims=True)
        cand = jnp.where(d == rowmax, iota, jnp.int32(2**30))
        amin = jnp.min(cand, axis=1, keepdims=True)
        idx_ref[0, :, pl.ds(t, 1)] = amin
        d = jnp.where(cand == amin, NEG, d)


def _topk_from_dist(dist):
    return pl.pallas_call(
        _topk_body,
        grid=(B, NR),
        in_specs=[pl.BlockSpec((1, R, N), lambda b, r: (b, r, 0))],
        out_specs=pl.BlockSpec((1, R, K), lambda b, r: (b, r, 0)),
        out_shape=jax.ShapeDtypeStruct((B, N, K), jnp.int32),
    )(dist)


def _knn_idx(h):
    inner = -2.0 * jnp.einsum('bcn,bcm->bnm', h, h)
    xx = jnp.sum(h * h, axis=1)
    dist = -xx[:, :, None] - inner - xx[:, None, :]
    return _topk_from_dist(dist)


def _conv_body(h_ref, w_ref, y_ref):
    y_ref[0] = jnp.dot(w_ref[...], h_ref[0], preferred_element_type=jnp.float32)


def _conv(h, W):
    return pl.pallas_call(
        _conv_body,
        grid=(B,),
        in_specs=[
            pl.BlockSpec((1, C, N), lambda b: (b, 0, 0)),
            pl.BlockSpec((C, C), lambda b: (0, 0)),
        ],
        out_specs=pl.BlockSpec((1, C, N), lambda b: (b, 0, 0)),
        out_shape=jax.ShapeDtypeStruct((B, C, N), jnp.float32),
    )(h, W)


@functools.partial(
    pl.kernel,
    mesh=plsc.VectorSubcoreMesh(core_axis_name="c", subcore_axis_name="s"),
    compiler_params=pltpu.CompilerParams(needs_layout_passes=False),
    out_type=(
        jax.ShapeDtypeStruct((B, 2, C * HALF), jnp.float32),
        jax.ShapeDtypeStruct((32, 2, C * 16), jnp.float32),
    ),
    scratch_types=[
        pltpu.VMEM((C * N,), jnp.float32),
        pltpu.VMEM((K * HALF,), jnp.int32),
        pltpu.VMEM((C * HALF,), jnp.float32),
        pltpu.VMEM((C * 16,), jnp.float32),
        pltpu.VMEM((C * 16,), jnp.float32),
    ],
)
def _sc_gathermax(y_hbm, idx_hbm, m_hbm, sp_hbm, yv, idxv, mv, s1v, s2v):
    b = lax.axis_index("s")
    half = lax.axis_index("c")
    wid = b * 2 + half
    pltpu.sync_copy(y_hbm.at[b], yv)
    pltpu.sync_copy(idx_hbm.at[b, half], idxv)

    def z_body(c, carry):
        off = pl.multiple_of(c * 16, 16)
        s1v[pl.ds(off, 16)] = jnp.zeros((16,), jnp.float32)
        s2v[pl.ds(off, 16)] = jnp.zeros((16,), jnp.float32)
        return carry

    lax.fori_loop(0, C, z_body, 0)

    def n0_body(n0, carry):
        nb = pl.multiple_of(n0 * 16, 16)
        ridx = [idxv[pl.ds(k * HALF + nb, 16)] for k in range(K)]

        def c_body(c, inner):
            cbase = jnp.full((16,), c * N, jnp.int32)
            acc = jnp.full((16,), NEG, jnp.float32)
            s1 = jnp.zeros((16,), jnp.float32)
            s2 = jnp.zeros((16,), jnp.float32)
            for k in range(K):
                v = plsc.load_gather(yv, [cbase + ridx[k]])
                acc = jnp.maximum(acc, v)
                s1 = s1 + v
                s2 = s2 + v * v
            mv[pl.ds(pl.multiple_of(c * HALF, 16) + nb, 16)] = acc
            soff = pl.multiple_of(c * 16, 16)
            s1v[pl.ds(soff, 16)] = s1v[pl.ds(soff, 16)] + s1
            s2v[pl.ds(soff, 16)] = s2v[pl.ds(soff, 16)] + s2
            return inner

        return lax.fori_loop(0, C, c_body, carry)

    lax.fori_loop(0, HALF // 16, n0_body, 0)
    pltpu.sync_copy(mv, m_hbm.at[b, half])
    pltpu.sync_copy(s1v, sp_hbm.at[wid, 0])
    pltpu.sync_copy(s2v, sp_hbm.at[wid, 1])


NTOT = float(B * N * K)


def _norm_body(m_ref, s1_ref, s2_ref, g_ref, b_ref, o_ref):
    s1 = jnp.sum(s1_ref[...], axis=1, keepdims=True)
    s2 = jnp.sum(s2_ref[...], axis=1, keepdims=True)
    mean = s1 / NTOT
    var = s2 / NTOT - mean * mean
    sc = g_ref[...] / jnp.sqrt(var + EPS)
    sh = b_ref[...] - mean * sc
    o_ref[0] = jnp.maximum(m_ref[0] * sc + sh, 0.0)


def _norm(m, s1p, s2p, g, b):
    return pl.pallas_call(
        _norm_body,
        grid=(B,),
        in_specs=[
            pl.BlockSpec((1, C, N), lambda i: (i, 0, 0)),
            pl.BlockSpec((C, 512), lambda i: (0, 0)),
            pl.BlockSpec((C, 512), lambda i: (0, 0)),
            pl.BlockSpec((C, 1), lambda i: (0, 0)),
            pl.BlockSpec((C, 1), lambda i: (0, 0)),
        ],
        out_specs=pl.BlockSpec((1, C, N), lambda i: (i, 0, 0)),
        out_shape=jax.ShapeDtypeStruct((B, C, N), jnp.float32),
    )(m, s1p, s2p, g, b)


def _layer5_body(h_ref, w_ref, cm_ref, s1_ref, s2_ref):
    z = jnp.dot(w_ref[...], h_ref[0], preferred_element_type=jnp.float32)
    cm_ref[0] = jnp.max(z, axis=1, keepdims=True)
    s1_ref[0] = jnp.sum(z, axis=1, keepdims=True)
    s2_ref[0] = jnp.sum(z * z, axis=1, keepdims=True)


def _layer5(h, W5):
    shp = jax.ShapeDtypeStruct((B, 1024, 1), jnp.float32)
    return pl.pallas_call(
        _layer5_body,
        grid=(B,),
        in_specs=[
            pl.BlockSpec((1, C, N), lambda b: (b, 0, 0)),
            pl.BlockSpec((1024, C), lambda b: (0, 0)),
        ],
        out_specs=[pl.BlockSpec((1, 1024, 1), lambda b: (b, 0, 0))] * 3,
        out_shape=[shp, shp, shp],
    )(h, W5)


def _head_body(cm_ref, s1_ref, s2_ref, g5_ref, b5_ref, w6_ref, g6_ref, b6_ref,
               w7_ref, g7_ref, b7_ref, w8_ref, bias8_ref, o_ref):
    n5 = float(B * N)
    s1 = jnp.sum(s1_ref[...], axis=0, keepdims=True)
    s2 = jnp.sum(s2_ref[...], axis=0, keepdims=True)
    mean = s1 / n5
    var = s2 / n5 - mean * mean
    sc5 = g5_ref[...] / jnp.sqrt(var + EPS)
    sh5 = b5_ref[...] - mean * sc5
    h5 = jnp.maximum(cm_ref[...] * sc5 + sh5, 0.0)

    a = jnp.dot(h5, w6_ref[...], preferred_element_type=jnp.float32)
    m6 = jnp.mean(a, axis=0, keepdims=True)
    v6 = jnp.mean((a - m6) * (a - m6), axis=0, keepdims=True)
    h6 = jnp.maximum((a - m6) / jnp.sqrt(v6 + EPS) * g6_ref[...] + b6_ref[...], 0.0)

    a = jnp.dot(h6, w7_ref[...], preferred_element_type=jnp.float32)
    m7 = jnp.mean(a, axis=0, keepdims=True)
    v7 = jnp.mean((a - m7) * (a - m7), axis=0, keepdims=True)
    h7 = jnp.maximum((a - m7) / jnp.sqrt(v7 + EPS) * g7_ref[...] + b7_ref[...], 0.0)

    o_ref[...] = jnp.maximum(
        jnp.dot(h7, w8_ref[...], preferred_element_type=jnp.float32)
        + bias8_ref[...], 0.0)


def _head(cm, s1, s2, g5, b5, W6T, g6, b6, W7T, g7, b7, W8T, bias8):
    return pl.pallas_call(
        _head_body,
        out_shape=jax.ShapeDtypeStruct((B, 40), jnp.float32),
    )(cm, s1, s2, g5, b5, W6T, g6, b6, W7T, g7, b7, W8T, bias8)


def _edge_layer_exact(h, W, g, b):
    idx = _knn_idx(h)
    x_t = jnp.transpose(h, (0, 2, 1))
    feat = jax.vmap(lambda xt, id_: xt[id_])(x_t, idx)
    feat = jnp.transpose(feat, (0, 3, 2, 1))
    z = jnp.einsum('oc,bckn->bokn', W, feat)
    mean = jnp.mean(z, axis=(0, 2, 3), keepdims=True)
    var = jnp.var(z, axis=(0, 2, 3), keepdims=True)
    zn = (z - mean) / jnp.sqrt(var + EPS) * g[None, :, None, None] + b[None, :, None, None]
    return jnp.max(jax.nn.relu(zn), axis=2)


def _edge_layer(h, W, g, b):
    idx = _knn_idx(h)
    idx4 = jnp.transpose(idx.reshape(B, 2, HALF, K), (0, 1, 3, 2)).reshape(B, 2, K * HALF)
    y = _conv(h, W)
    m4, sp = _sc_gathermax(y.reshape(B, C * N), idx4)
    m = jnp.transpose(m4.reshape(B, 2, C, HALF), (0, 2, 1, 3)).reshape(B, C, N)
    s1p = jnp.transpose(sp[:, 0].reshape(32, C, 16), (1, 0, 2)).reshape(C, 512)
    s2p = jnp.transpose(sp[:, 1].reshape(32, C, 16), (1, 0, 2)).reshape(C, 512)
    return _norm(m, s1p, s2p, g.reshape(C, 1), b.reshape(C, 1))


def kernel(x, W1, g1, b1, W2, g2, b2, W3, g3, b3, W4, g4, b4, W5, g5, b5,
           W6, bias6, g6, b6, W7, bias7, g7, b7, W8, bias8):
    h = _edge_layer_exact(x, W1, g1, b1)
    h = _edge_layer_exact(h, W2, g2, b2)
    h = _edge_layer_exact(h, W3, g3, b3)
    h = _edge_layer(h, W4, g4, b4)
    cm, s1, s2 = _layer5(h, W5)
    out = _head(
        cm.reshape(B, 1024), s1.reshape(B, 1024), s2.reshape(B, 1024),
        g5.reshape(1, 1024), b5.reshape(1, 1024),
        W6.T, g6.reshape(1, 512), (b6 + 0.0 * bias6).reshape(1, 512),
        W7.T, g7.reshape(1, 128), (b7 + 0.0 * bias7).reshape(1, 128),
        W8.T, bias8.reshape(1, 40))
    return out
